# baseline (device time: 130590 ns/iter reference)
import functools

import jax
import jax.numpy as jnp
from jax import lax
from jax.experimental import pallas as pl
from jax.experimental.pallas import tpu as pltpu

N_DEV = 4
B, SQ, SKV, HQ, DH = 2, 512, 512, 8, 64
DM = 768
DQ = HQ * DH


def kernel(x, Wq, K_ext, V_ext, Wo):
    my = lax.axis_index("i")
    K = lax.dynamic_slice_in_dim(K_ext, my * HQ, HQ, axis=2)
    V = lax.dynamic_slice_in_dim(V_ext, my * HQ, HQ, axis=2)
    Kt = jnp.transpose(K, (0, 2, 1, 3)).reshape(B * HQ, SKV, DH)
    Vt = jnp.transpose(V, (0, 2, 1, 3)).reshape(B * HQ, SKV, DH)

    def body(x_ref, wq_ref, kt_ref, vt_ref, wo_ref, out_ref,
             ctx_ref, comm_ref, send_sems, recv_sems):
        my_pos = lax.axis_index("i")
        left = lax.rem(my_pos + N_DEV - 1, N_DEV)
        right = lax.rem(my_pos + 1, N_DEV)

        barrier_sem = pltpu.get_barrier_semaphore()
        for nbr in (left, right):
            pl.semaphore_signal(barrier_sem, inc=1, device_id=(nbr,),
                                device_id_type=pl.DeviceIdType.MESH)
        pl.semaphore_wait(barrier_sem, 2)

        x2 = x_ref[:].reshape(B * SQ, DM)
        q_all = jnp.dot(x2, wq_ref[:], preferred_element_type=jnp.float32)

        qi = lax.broadcasted_iota(jnp.int32, (SQ, SKV), 0)
        ki = lax.broadcasted_iota(jnp.int32, (SQ, SKV), 1)
        mask = (jnp.abs(qi - ki) <= 128) | (ki < 32) | (qi < 32)

        for b in range(B):
            for h in range(HQ):
                bh = b * HQ + h
                q = q_all[b * SQ:(b + 1) * SQ, h * DH:(h + 1) * DH]
                k = kt_ref[bh]
                s = lax.dot_general(q, k, (((1,), (1,)), ((), ())),
                                    preferred_element_type=jnp.float32)
                s = jnp.where(mask, s * 0.125, -1e9)
                m = jnp.max(s, axis=1, keepdims=True)
                w = jnp.exp(s - m)
                w = w / jnp.sum(w, axis=1, keepdims=True)
                ctx = jnp.dot(w, vt_ref[bh],
                              preferred_element_type=jnp.float32)
                ctx_ref[b * SQ:(b + 1) * SQ, h * DH:(h + 1) * DH] = ctx

        partial = jnp.dot(ctx_ref[:], wo_ref[:],
                          preferred_element_type=jnp.float32)
        comm_ref[0, :, :] = partial
        acc = partial

        for hop in range(N_DEV - 1):
            rdma = pltpu.make_async_remote_copy(
                src_ref=comm_ref.at[hop],
                dst_ref=comm_ref.at[hop + 1],
                send_sem=send_sems.at[hop],
                recv_sem=recv_sems.at[hop],
                device_id=(right,),
                device_id_type=pl.DeviceIdType.MESH,
            )
            rdma.start()
            rdma.wait()
            acc = acc + comm_ref[hop + 1, :, :]

        out_ref[:] = acc.reshape(B, SQ, DM)

        @functools.partial(pl.run_scoped,
                           exit_sem=pltpu.SemaphoreType.REGULAR)
        def _(exit_sem):
            for nbr in (left, right):
                pl.semaphore_signal(exit_sem, inc=1, device_id=(nbr,),
                                    device_id_type=pl.DeviceIdType.MESH)
            pl.semaphore_wait(exit_sem, 2)

    return pl.pallas_call(
        body,
        out_shape=jax.ShapeDtypeStruct((B, SQ, DM), jnp.float32),
        in_specs=[pl.BlockSpec(memory_space=pltpu.VMEM)] * 5,
        out_specs=pl.BlockSpec(memory_space=pltpu.VMEM),
        scratch_shapes=[
            pltpu.VMEM((B * SQ, DQ), jnp.float32),
            pltpu.VMEM((N_DEV, B * SQ, DM), jnp.float32),
            pltpu.SemaphoreType.DMA((N_DEV - 1,)),
            pltpu.SemaphoreType.DMA((N_DEV - 1,)),
        ],
        compiler_params=pltpu.CompilerParams(collective_id=0),
    )(x, Wq, Kt, Vt, Wo)


# device time: 62100 ns/iter; 2.1029x vs baseline; 2.1029x over previous
import functools

import jax
import jax.numpy as jnp
from jax import lax
from jax.experimental import pallas as pl
from jax.experimental.pallas import tpu as pltpu

N_DEV = 4
B, SQ, SKV, HQ, DH = 2, 512, 512, 8, 64
DM = 768
DQ = HQ * DH
ROWS = B * SQ
CH = ROWS // N_DEV


def kernel(x, Wq, K_ext, V_ext, Wo):
    my = lax.axis_index("i")
    K = lax.dynamic_slice_in_dim(K_ext, my * HQ, HQ, axis=2)
    V = lax.dynamic_slice_in_dim(V_ext, my * HQ, HQ, axis=2)
    Kt = jnp.transpose(K, (0, 2, 1, 3)).reshape(B * HQ, SKV, DH)
    Vt = jnp.transpose(V, (0, 2, 1, 3)).reshape(B * HQ, SKV, DH)
    x2 = x.reshape(ROWS, DM)

    def body(x_ref, wq_ref, kt_ref, vt_ref, wo_ref, out_ref,
             ctx_ref, part_ref, rs_ref, send_sems, rs_recv_sems,
             ag_recv_sems):
        me = lax.axis_index("i")

        barrier_sem = pltpu.get_barrier_semaphore()
        for r in range(1, N_DEV):
            pl.semaphore_signal(barrier_sem, inc=1,
                                device_id=(lax.rem(me + r, N_DEV),),
                                device_id_type=pl.DeviceIdType.MESH)
        pl.semaphore_wait(barrier_sem, N_DEV - 1)

        q_all = jnp.dot(x_ref[:], wq_ref[:],
                        preferred_element_type=jnp.float32)

        qi = lax.broadcasted_iota(jnp.int32, (SQ, SKV), 0)
        ki = lax.broadcasted_iota(jnp.int32, (SQ, SKV), 1)
        mask = (jnp.abs(qi - ki) <= 128) | (ki < 32) | (qi < 32)

        for b in range(B):
            for h in range(HQ):
                bh = b * HQ + h
                q = q_all[b * SQ:(b + 1) * SQ, h * DH:(h + 1) * DH]
                k = kt_ref[bh]
                s = lax.dot_general(q, k, (((1,), (1,)), ((), ())),
                                    preferred_element_type=jnp.float32)
                s = jnp.where(mask, s * 0.125, -1e9)
                m = jnp.max(s, axis=1, keepdims=True)
                w = jnp.exp(s - m)
                w = w / jnp.sum(w, axis=1, keepdims=True)
                ctx = jnp.dot(w, vt_ref[bh],
                              preferred_element_type=jnp.float32)
                ctx_ref[b * SQ:(b + 1) * SQ, h * DH:(h + 1) * DH] = ctx

        part_ref[:] = jnp.dot(ctx_ref[:], wo_ref[:],
                              preferred_element_type=jnp.float32)

        rs = []
        for r in range(1, N_DEV):
            p = lax.rem(me + r, N_DEV)
            rdma = pltpu.make_async_remote_copy(
                src_ref=part_ref.at[pl.ds(p * CH, CH), :],
                dst_ref=rs_ref.at[r - 1],
                send_sem=send_sems.at[r - 1],
                recv_sem=rs_recv_sems.at[r - 1],
                device_id=(p,),
                device_id_type=pl.DeviceIdType.MESH,
            )
            rdma.start()
            rs.append(rdma)
        for rdma in rs:
            rdma.wait_recv()

        red = (part_ref[pl.ds(me * CH, CH), :]
               + rs_ref[0] + rs_ref[1] + rs_ref[2])
        out_ref[pl.ds(me * CH, CH), :] = red

        for rdma in rs:
            rdma.wait_send()

        ag = []
        for r in range(1, N_DEV):
            p = lax.rem(me + r, N_DEV)
            rdma = pltpu.make_async_remote_copy(
                src_ref=out_ref.at[pl.ds(me * CH, CH), :],
                dst_ref=out_ref.at[pl.ds(me * CH, CH), :],
                send_sem=send_sems.at[r - 1],
                recv_sem=ag_recv_sems.at[N_DEV - 1 - r],
                device_id=(p,),
                device_id_type=pl.DeviceIdType.MESH,
            )
            rdma.start()
            ag.append(rdma)
        for rdma in ag:
            rdma.wait_recv()
        for rdma in ag:
            rdma.wait_send()

        @functools.partial(pl.run_scoped,
                           exit_sem=pltpu.SemaphoreType.REGULAR)
        def _(exit_sem):
            for r in range(1, N_DEV):
                pl.semaphore_signal(exit_sem, inc=1,
                                    device_id=(lax.rem(me + r, N_DEV),),
                                    device_id_type=pl.DeviceIdType.MESH)
            pl.semaphore_wait(exit_sem, N_DEV - 1)

    out = pl.pallas_call(
        body,
        out_shape=jax.ShapeDtypeStruct((ROWS, DM), jnp.float32),
        in_specs=[pl.BlockSpec(memory_space=pltpu.VMEM)] * 5,
        out_specs=pl.BlockSpec(memory_space=pltpu.VMEM),
        scratch_shapes=[
            pltpu.VMEM((ROWS, DQ), jnp.float32),
            pltpu.VMEM((ROWS, DM), jnp.float32),
            pltpu.VMEM((N_DEV - 1, CH, DM), jnp.float32),
            pltpu.SemaphoreType.DMA((N_DEV - 1,)),
            pltpu.SemaphoreType.DMA((N_DEV - 1,)),
            pltpu.SemaphoreType.DMA((N_DEV - 1,)),
        ],
        compiler_params=pltpu.CompilerParams(collective_id=0),
    )(x2, Wq, Kt, Vt, Wo)
    return out.reshape(B, SQ, DM)


# device time: 57572 ns/iter; 2.2683x vs baseline; 1.0786x over previous
import functools

import jax
import jax.numpy as jnp
from jax import lax
from jax.experimental import pallas as pl
from jax.experimental.pallas import tpu as pltpu

N_DEV = 4
B, SQ, SKV, HQ, DH = 2, 512, 512, 8, 64
DM = 768
DQ = HQ * DH
ROWS = B * SQ
CH = ROWS // N_DEV


def kernel(x, Wq, K_ext, V_ext, Wo):
    my = lax.axis_index("i")
    K = lax.dynamic_slice_in_dim(K_ext, my * HQ, HQ, axis=2)
    V = lax.dynamic_slice_in_dim(V_ext, my * HQ, HQ, axis=2)
    Kt = jnp.transpose(K, (0, 2, 1, 3)).reshape(B * HQ, SKV, DH)
    Vt = jnp.transpose(V, (0, 2, 1, 3)).reshape(B * HQ, SKV, DH)
    x2 = x.reshape(ROWS, DM)

    def body(x_ref, wq_ref, kt_ref, vt_ref, wo_ref, out_ref,
             ctx_ref, part_ref, rs_ref, send_sems, rs_recv_sems,
             ag_recv_sems):
        me = lax.axis_index("i")

        barrier_sem = pltpu.get_barrier_semaphore()
        for r in range(1, N_DEV):
            pl.semaphore_signal(barrier_sem, inc=1,
                                device_id=(lax.rem(me + r, N_DEV),),
                                device_id_type=pl.DeviceIdType.MESH)
        pl.semaphore_wait(barrier_sem, N_DEV - 1)

        ki = lax.broadcasted_iota(jnp.int32, (CH, SKV), 1)
        qi0 = lax.broadcasted_iota(jnp.int32, (CH, SKV), 0)

        def compute_chunk(c):
            b = lax.div(c, 2)
            qoff = lax.rem(c, 2) * CH
            xq = x_ref[pl.ds(c * CH, CH), :]
            q_c = jnp.dot(xq, wq_ref[:],
                          preferred_element_type=jnp.float32)
            qi = qi0 + qoff
            mask = (jnp.abs(qi - ki) <= 128) | (ki < 32) | (qi < 32)
            for h in range(HQ):
                bh = b * HQ + h
                q = q_c[:, h * DH:(h + 1) * DH]
                k = kt_ref[bh]
                s = lax.dot_general(q, k, (((1,), (1,)), ((), ())),
                                    preferred_element_type=jnp.float32)
                s = jnp.where(mask, s * 0.125, -1e9)
                m = jnp.max(s, axis=1, keepdims=True)
                w = jnp.exp(s - m)
                w = w / jnp.sum(w, axis=1, keepdims=True)
                ctx = jnp.dot(w, vt_ref[bh],
                              preferred_element_type=jnp.float32)
                ctx_ref[:, h * DH:(h + 1) * DH] = ctx
            return jnp.dot(ctx_ref[:], wo_ref[:],
                           preferred_element_type=jnp.float32)

        rs = []
        for r in range(1, N_DEV):
            c = lax.rem(me + r, N_DEV)
            part_ref[r - 1, :, :] = compute_chunk(c)
            rdma = pltpu.make_async_remote_copy(
                src_ref=part_ref.at[r - 1],
                dst_ref=rs_ref.at[r - 1],
                send_sem=send_sems.at[r - 1],
                recv_sem=rs_recv_sems.at[r - 1],
                device_id=(c,),
                device_id_type=pl.DeviceIdType.MESH,
            )
            rdma.start()
            rs.append(rdma)

        own = compute_chunk(me)
        for rdma in rs:
            rdma.wait_recv()
        red = own + rs_ref[0] + rs_ref[1] + rs_ref[2]
        out_ref[pl.ds(me * CH, CH), :] = red
        for rdma in rs:
            rdma.wait_send()

        ag = []
        for r in range(1, N_DEV):
            p = lax.rem(me + r, N_DEV)
            rdma = pltpu.make_async_remote_copy(
                src_ref=out_ref.at[pl.ds(me * CH, CH), :],
                dst_ref=out_ref.at[pl.ds(me * CH, CH), :],
                send_sem=send_sems.at[r - 1],
                recv_sem=ag_recv_sems.at[N_DEV - 1 - r],
                device_id=(p,),
                device_id_type=pl.DeviceIdType.MESH,
            )
            rdma.start()
            ag.append(rdma)
        for rdma in ag:
            rdma.wait_recv()
        for rdma in ag:
            rdma.wait_send()

        @functools.partial(pl.run_scoped,
                           exit_sem=pltpu.SemaphoreType.REGULAR)
        def _(exit_sem):
            for r in range(1, N_DEV):
                pl.semaphore_signal(exit_sem, inc=1,
                                    device_id=(lax.rem(me + r, N_DEV),),
                                    device_id_type=pl.DeviceIdType.MESH)
            pl.semaphore_wait(exit_sem, N_DEV - 1)

    out = pl.pallas_call(
        body,
        out_shape=jax.ShapeDtypeStruct((ROWS, DM), jnp.float32),
        in_specs=[pl.BlockSpec(memory_space=pltpu.VMEM)] * 5,
        out_specs=pl.BlockSpec(memory_space=pltpu.VMEM),
        scratch_shapes=[
            pltpu.VMEM((CH, DQ), jnp.float32),
            pltpu.VMEM((N_DEV - 1, CH, DM), jnp.float32),
            pltpu.VMEM((N_DEV - 1, CH, DM), jnp.float32),
            pltpu.SemaphoreType.DMA((N_DEV - 1,)),
            pltpu.SemaphoreType.DMA((N_DEV - 1,)),
            pltpu.SemaphoreType.DMA((N_DEV - 1,)),
        ],
        compiler_params=pltpu.CompilerParams(collective_id=0),
    )(x2, Wq, Kt, Vt, Wo)
    return out.reshape(B, SQ, DM)


# device time: 40957 ns/iter; 3.1885x vs baseline; 1.4057x over previous
import functools

import jax
import jax.numpy as jnp
from jax import lax
from jax.experimental import pallas as pl
from jax.experimental.pallas import tpu as pltpu

N_DEV = 4
B, SQ, SKV, HQ, DH = 2, 512, 512, 8, 64
DM = 768
DQ = HQ * DH
ROWS = B * SQ
CH = ROWS // N_DEV


def kernel(x, Wq, K_ext, V_ext, Wo):
    my = lax.axis_index("i")
    K = lax.dynamic_slice_in_dim(K_ext, my * HQ, HQ, axis=2)
    V = lax.dynamic_slice_in_dim(V_ext, my * HQ, HQ, axis=2)
    Kt = jnp.transpose(K, (0, 2, 1, 3)).reshape(B * HQ, SKV, DH)
    Vt = jnp.transpose(V, (0, 2, 1, 3)).reshape(B * HQ, SKV, DH)
    x2 = x.reshape(ROWS, DM)

    def body(x_ref, wq_ref, kt_ref, vt_ref, wo_ref, out_ref,
             ctx_ref, part_ref, rs_ref, agsrc_ref, ag_ref,
             send_sems, rs_recv_sems, ag_recv_sems):
        me = lax.axis_index("i")

        barrier_sem = pltpu.get_barrier_semaphore()
        for r in range(1, N_DEV):
            pl.semaphore_signal(barrier_sem, inc=1,
                                device_id=(lax.rem(me + r, N_DEV),),
                                device_id_type=pl.DeviceIdType.MESH)
        pl.semaphore_wait(barrier_sem, N_DEV - 1)

        ki = lax.broadcasted_iota(jnp.int32, (CH, SKV), 1)
        qi0 = lax.broadcasted_iota(jnp.int32, (CH, SKV), 0)

        def compute_chunk(c):
            b = lax.div(c, 2)
            qoff = lax.rem(c, 2) * CH
            xq = x_ref[pl.ds(c * CH, CH), :]
            q_c = jnp.dot(xq, wq_ref[:],
                          preferred_element_type=jnp.float32)
            qi = qi0 + qoff
            mask = (jnp.abs(qi - ki) <= 128) | (ki < 32) | (qi < 32)
            for h in range(HQ):
                bh = b * HQ + h
                q = q_c[:, h * DH:(h + 1) * DH]
                k = kt_ref[bh]
                s = lax.dot_general(q, k, (((1,), (1,)), ((), ())),
                                    preferred_element_type=jnp.float32)
                s = jnp.where(mask, s * 0.125, -1e9)
                m = jnp.max(s, axis=1, keepdims=True)
                w = jnp.exp(s - m)
                w = w / jnp.sum(w, axis=1, keepdims=True)
                ctx = jnp.dot(w, vt_ref[bh],
                              preferred_element_type=jnp.float32)
                ctx_ref[:, h * DH:(h + 1) * DH] = ctx
            return jnp.dot(ctx_ref[:], wo_ref[:],
                           preferred_element_type=jnp.float32)

        rs = []
        for r in range(1, N_DEV):
            c = lax.rem(me + r, N_DEV)
            part_ref[r - 1, :, :] = compute_chunk(c).astype(jnp.bfloat16)
            rdma = pltpu.make_async_remote_copy(
                src_ref=part_ref.at[r - 1],
                dst_ref=rs_ref.at[r - 1],
                send_sem=send_sems.at[r - 1],
                recv_sem=rs_recv_sems.at[r - 1],
                device_id=(c,),
                device_id_type=pl.DeviceIdType.MESH,
            )
            rdma.start()
            rs.append(rdma)

        own = compute_chunk(me)
        for rdma in rs:
            rdma.wait_recv()
        red = (own
               + rs_ref[0].astype(jnp.float32)
               + rs_ref[1].astype(jnp.float32)
               + rs_ref[2].astype(jnp.float32))
        out_ref[pl.ds(me * CH, CH), :] = red
        agsrc_ref[:] = red.astype(jnp.bfloat16)
        for rdma in rs:
            rdma.wait_send()

        ag = []
        for r in range(1, N_DEV):
            p = lax.rem(me + r, N_DEV)
            rdma = pltpu.make_async_remote_copy(
                src_ref=agsrc_ref,
                dst_ref=ag_ref.at[N_DEV - 1 - r],
                send_sem=send_sems.at[r - 1],
                recv_sem=ag_recv_sems.at[N_DEV - 1 - r],
                device_id=(p,),
                device_id_type=pl.DeviceIdType.MESH,
            )
            rdma.start()
            ag.append(rdma)
        for rdma in ag:
            rdma.wait_recv()
        for j in range(N_DEV - 1):
            p = lax.rem(me + j + 1, N_DEV)
            out_ref[pl.ds(p * CH, CH), :] = ag_ref[j].astype(jnp.float32)
        for rdma in ag:
            rdma.wait_send()

        @functools.partial(pl.run_scoped,
                           exit_sem=pltpu.SemaphoreType.REGULAR)
        def _(exit_sem):
            for r in range(1, N_DEV):
                pl.semaphore_signal(exit_sem, inc=1,
                                    device_id=(lax.rem(me + r, N_DEV),),
                                    device_id_type=pl.DeviceIdType.MESH)
            pl.semaphore_wait(exit_sem, N_DEV - 1)

    out = pl.pallas_call(
        body,
        out_shape=jax.ShapeDtypeStruct((ROWS, DM), jnp.float32),
        in_specs=[pl.BlockSpec(memory_space=pltpu.VMEM)] * 5,
        out_specs=pl.BlockSpec(memory_space=pltpu.VMEM),
        scratch_shapes=[
            pltpu.VMEM((CH, DQ), jnp.float32),
            pltpu.VMEM((N_DEV - 1, CH, DM), jnp.bfloat16),
            pltpu.VMEM((N_DEV - 1, CH, DM), jnp.bfloat16),
            pltpu.VMEM((CH, DM), jnp.bfloat16),
            pltpu.VMEM((N_DEV - 1, CH, DM), jnp.bfloat16),
            pltpu.SemaphoreType.DMA((N_DEV - 1,)),
            pltpu.SemaphoreType.DMA((N_DEV - 1,)),
            pltpu.SemaphoreType.DMA((N_DEV - 1,)),
        ],
        compiler_params=pltpu.CompilerParams(collective_id=0),
    )(x2, Wq, Kt, Vt, Wo)
    return out.reshape(B, SQ, DM)
